# baseline (speedup 1.0000x reference)
import functools
import math

import jax
import jax.numpy as jnp
from jax import lax
from jax.experimental import pallas as pl
from jax.experimental.pallas import tpu as pltpu
from jax.experimental.pallas import tpu_sc as plsc

_SMOOTHING = 0.1
_CONFIDENCE = 1.0 - _SMOOTHING

_N_SC = 128
_NW = 32
_NC = 2
_CHUNK = 20000


def _tc_kernel(inp_ref, tgt_ref, mask_ref, out_ref, num_ref, den_ref, *,
               nblocks, V, C):
    b = pl.program_id(0)

    @pl.when(b == 0)
    def _init():
        num_ref[:, :] = jnp.zeros_like(num_ref)
        den_ref[:, :] = jnp.zeros_like(den_ref)

    eps = _SMOOTHING / (V - 1)
    x = inp_ref[:, :]
    col = jax.lax.broadcasted_iota(jnp.int32, x.shape, 1)
    x = jnp.where(col < V, x, 0.0)
    hit = col == tgt_ref[:, :]
    asum = jnp.sum(x, axis=1, keepdims=True)
    atgt = jnp.sum(jnp.where(hit, x, 0.0), axis=1, keepdims=True)
    m = mask_ref[:, :]
    kl = C - eps * asum - (_CONFIDENCE - eps) * atgt
    num_ref[:, :] += jnp.sum(m * kl, keepdims=True)
    den_ref[:, :] += jnp.sum(m, keepdims=True)

    @pl.when(b == nblocks - 1)
    def _finish():
        out_ref[:, 0:1] = num_ref[:, :]
        out_ref[:, 1:2] = den_ref[:, :]


def _sc_kernel(flat_ref, tgt_ref, mask_ref, out_ref, buf0, buf1, idx_v, m_v,
               res_v, sem0, sem1, *, V, C, r_per_w, row0):
    eps = _SMOOTHING / (V - 1)
    wid = lax.axis_index("s") * _NC + lax.axis_index("c")
    base_row = row0 + wid * r_per_w
    base_off = base_row * V
    nchunks = r_per_w * (V // _CHUNK)
    cpr = V // _CHUNK

    pltpu.sync_copy(tgt_ref.at[wid], idx_v)
    pltpu.sync_copy(mask_ref.at[wid], m_v)

    sems = [sem0, sem1]
    bufs = [buf0, buf1]

    def start(k):
        return pltpu.async_copy(
            flat_ref.at[pl.ds(base_off + k * _CHUNK, _CHUNK)],
            bufs[k % 2], sems[k % 2])

    handles = [start(0), None]
    tvec = idx_v[...]
    mvec = m_v[...]
    lane = lax.iota(jnp.int32, 16)
    num_vec = jnp.zeros((16,), jnp.float32)
    den_vec = jnp.zeros((16,), jnp.float32)
    for r in range(r_per_w):
        t_r = tvec[r]
        m_r = mvec[r]
        s_r = jnp.zeros((16,), jnp.float32)
        g_r = jnp.zeros((16,), jnp.float32)
        for c in range(cpr):
            k = r * cpr + c
            if k + 1 < nchunks:
                handles[(k + 1) % 2] = start(k + 1)
            handles[k % 2].wait()
            cbuf = bufs[k % 2]

            def body(i, accs):
                a0, a1, a2, a3 = accs
                o = pl.multiple_of(i * 160, 16)
                a0 += cbuf[pl.ds(o, 16)]
                a1 += cbuf[pl.ds(o + 16, 16)]
                a2 += cbuf[pl.ds(o + 32, 16)]
                a3 += cbuf[pl.ds(o + 48, 16)]
                a0 += cbuf[pl.ds(o + 64, 16)]
                a1 += cbuf[pl.ds(o + 80, 16)]
                a2 += cbuf[pl.ds(o + 96, 16)]
                a3 += cbuf[pl.ds(o + 112, 16)]
                a0 += cbuf[pl.ds(o + 128, 16)]
                a1 += cbuf[pl.ds(o + 144, 16)]
                return a0, a1, a2, a3

            z = jnp.zeros((16,), jnp.float32)
            a0, a1, a2, a3 = lax.fori_loop(0, _CHUNK // 160, body,
                                           (z, z, z, z))
            s_r = s_r + a0 + a1 + a2 + a3
            local = t_r - c * _CHUNK
            lidx = jnp.clip(local, 0, _CHUNK - 1)
            lstart = pl.multiple_of(lax.bitwise_and(lidx, -16), 16)
            tv = cbuf[pl.ds(lstart, 16)]
            inb = jnp.logical_and(local >= 0, local < _CHUNK)
            delta = jnp.where(inb, lidx - lstart, -1)
            hit = lane == jnp.broadcast_to(delta, (16,))
            g_r = g_r + jnp.where(hit, tv, 0.0)
        c_term = jnp.where(lane == 0, jnp.broadcast_to(m_r * C, (16,)), 0.0)
        num_vec = (num_vec + c_term
                   - (eps * m_r) * s_r
                   - ((_CONFIDENCE - eps) * m_r) * g_r)
        den_vec = den_vec + jnp.where(
            lane == 0, jnp.broadcast_to(m_r, (16,)), 0.0)
    res_v[0, :] = num_vec
    res_v[1, :] = den_vec
    pltpu.sync_copy(res_v, out_ref.at[wid])


def kernel(input, target, mask):
    S = input.shape[1]
    V = input.shape[-1]
    target = target[:, :S]
    mask = mask[:, :S]
    inp = input.reshape(-1, V)
    N = inp.shape[0]
    tgt = target.reshape(N).astype(jnp.int32)
    m = mask.reshape(N).astype(jnp.float32)

    eps = _SMOOTHING / (V - 1)
    C = (V - 1) * eps * math.log(eps) + _CONFIDENCE * math.log(_CONFIDENCE)

    n_tc = N - _N_SC
    r_per_w = _N_SC // _NW

    R = 32
    nblocks = n_tc // R
    tc_out = pl.pallas_call(
        functools.partial(_tc_kernel, nblocks=nblocks, V=V, C=C),
        grid=(nblocks,),
        in_specs=[
            pl.BlockSpec((R, V), lambda b: (b, 0)),
            pl.BlockSpec((R, 1), lambda b: (b, 0)),
            pl.BlockSpec((R, 1), lambda b: (b, 0)),
        ],
        out_specs=pl.BlockSpec((1, 2), lambda b: (0, 0)),
        out_shape=jax.ShapeDtypeStruct((1, 2), jnp.float32),
        scratch_shapes=[pltpu.VMEM((1, 1), jnp.float32),
                        pltpu.VMEM((1, 1), jnp.float32)],
    )(inp, tgt.reshape(N, 1), m.reshape(N, 1))

    flat = input.reshape(-1)
    lanepad = jnp.zeros((_NW, 16 - r_per_w), jnp.int32)
    tgt_sc = jnp.concatenate(
        [tgt[n_tc:].reshape(_NW, r_per_w), lanepad], axis=1)
    m_sc = jnp.concatenate(
        [m[n_tc:].reshape(_NW, r_per_w), lanepad.astype(jnp.float32)], axis=1)

    mesh = plsc.VectorSubcoreMesh(core_axis_name="c", subcore_axis_name="s")
    sc_fn = functools.partial(
        pl.kernel,
        mesh=mesh,
        out_type=jax.ShapeDtypeStruct((_NW, 2, 16), jnp.float32),
        scratch_types=[
            pltpu.VMEM((_CHUNK,), jnp.float32),
            pltpu.VMEM((_CHUNK,), jnp.float32),
            pltpu.VMEM((16,), jnp.int32),
            pltpu.VMEM((16,), jnp.float32),
            pltpu.VMEM((2, 16), jnp.float32),
            pltpu.SemaphoreType.DMA,
            pltpu.SemaphoreType.DMA,
        ],
    )(functools.partial(_sc_kernel, V=V, C=C, r_per_w=r_per_w, row0=n_tc))
    sc_out = sc_fn(flat, tgt_sc, m_sc)

    num = tc_out[0, 0] + jnp.sum(sc_out[:, 0, :])
    den = tc_out[0, 1] + jnp.sum(sc_out[:, 1, :])
    return num / den


# speedup vs baseline: 4.1161x; 4.1161x over previous
import functools
import math

import jax
import jax.numpy as jnp
from jax import lax
from jax.experimental import pallas as pl
from jax.experimental.pallas import tpu as pltpu
from jax.experimental.pallas import tpu_sc as plsc

_SMOOTHING = 0.1
_CONFIDENCE = 1.0 - _SMOOTHING

_N_SC = 128
_NW = 32
_NC = 2


def _tc_kernel(inp_ref, tgt_ref, mask_ref, out_ref, num_ref, den_ref, *,
               nblocks, V, C):
    b = pl.program_id(0)

    @pl.when(b == 0)
    def _init():
        num_ref[:, :] = jnp.zeros_like(num_ref)
        den_ref[:, :] = jnp.zeros_like(den_ref)

    eps = _SMOOTHING / (V - 1)
    x = inp_ref[:, :]
    col = jax.lax.broadcasted_iota(jnp.int32, x.shape, 1)
    x = jnp.where(col < V, x, 0.0)
    hit = col == tgt_ref[:, :]
    asum = jnp.sum(x, axis=1, keepdims=True)
    atgt = jnp.sum(jnp.where(hit, x, 0.0), axis=1, keepdims=True)
    m = mask_ref[:, :]
    kl = C - eps * asum - (_CONFIDENCE - eps) * atgt
    num_ref[:, :] += jnp.sum(m * kl, keepdims=True)
    den_ref[:, :] += jnp.sum(m, keepdims=True)

    @pl.when(b == nblocks - 1)
    def _finish():
        out_ref[:, 0:1] = num_ref[:, :]
        out_ref[:, 1:2] = den_ref[:, :]


_HALF = 50048
_CW = 2944
_NCH = 17


def _sc_kernel(inp_ref, tgt_ref, mask_ref, out_ref, buf0, buf1, idx_v, m_v,
               res_v, sem0, sem1, *, V, C, row0):
    eps = _SMOOTHING / (V - 1)
    wid = lax.axis_index("s") * _NC + lax.axis_index("c")
    g = wid // 2
    h = wid % 2
    grow = row0 + g * 8
    cbase = h * _HALF

    pltpu.sync_copy(tgt_ref.at[wid], idx_v)
    pltpu.sync_copy(mask_ref.at[wid], m_v)

    sems = [sem0, sem1]
    bufs = [buf0, buf1]
    row_ds = pl.ds(pl.multiple_of(grow, 8), 8)

    def start(k):
        col = pl.multiple_of(cbase + k * _CW, 128)
        return pltpu.async_copy(inp_ref.at[row_ds, pl.ds(col, _CW)],
                                bufs[k % 2], sems[k % 2])

    handles = [start(0), None]
    tvec = idx_v[...]
    mvec = m_v[...]
    lane = lax.iota(jnp.int32, 16)
    accs = tuple(jnp.zeros((16,), jnp.float32) for _ in range(8))
    gvec = [jnp.zeros((16,), jnp.float32) for _ in range(8)]
    thr_last = jnp.broadcast_to(V - cbase - (_NCH - 1) * _CW, (16,))
    for k in range(_NCH):
        if k + 1 < _NCH:
            handles[(k + 1) % 2] = start(k + 1)
        handles[k % 2].wait()
        cbuf = bufs[k % 2]

        if k < _NCH - 1:
            def body(i, a):
                o = pl.multiple_of(i * 16, 16)
                return tuple(a[r] + cbuf[r, pl.ds(o, 16)] for r in range(8))
        else:
            def body(i, a):
                o = pl.multiple_of(i * 16, 16)
                valid = (i * 16 + lane) < thr_last
                return tuple(
                    a[r] + jnp.where(valid, cbuf[r, pl.ds(o, 16)], 0.0)
                    for r in range(8))
        accs = lax.fori_loop(0, _CW // 16, body, accs)

        for r in range(8):
            local = tvec[r] - cbase - k * _CW
            lidx = jnp.clip(local, 0, _CW - 1)
            lstart = pl.multiple_of(lax.bitwise_and(lidx, -16), 16)
            tv = cbuf[r, pl.ds(lstart, 16)]
            inb = jnp.logical_and(local >= 0, local < _CW)
            delta = jnp.where(inb, lidx - lstart, -1)
            hit = lane == jnp.broadcast_to(delta, (16,))
            gvec[r] = gvec[r] + jnp.where(hit, tv, 0.0)

    h0 = jnp.where(h == 0, 1.0, 0.0)
    num_vec = jnp.zeros((16,), jnp.float32)
    den_vec = jnp.zeros((16,), jnp.float32)
    for r in range(8):
        m_r = mvec[r]
        num_vec = (num_vec
                   + jnp.where(lane == 0,
                               jnp.broadcast_to(m_r * C * h0, (16,)), 0.0)
                   - (eps * m_r) * accs[r]
                   - ((_CONFIDENCE - eps) * m_r) * gvec[r])
        den_vec = den_vec + jnp.where(
            lane == 0, jnp.broadcast_to(m_r * h0, (16,)), 0.0)
    res_v[0, :] = num_vec
    res_v[1, :] = den_vec
    pltpu.sync_copy(res_v, out_ref.at[wid])


def kernel(input, target, mask):
    S = input.shape[1]
    V = input.shape[-1]
    target = target[:, :S]
    mask = mask[:, :S]
    inp = input.reshape(-1, V)
    N = inp.shape[0]
    tgt = target.reshape(N).astype(jnp.int32)
    m = mask.reshape(N).astype(jnp.float32)

    eps = _SMOOTHING / (V - 1)
    C = (V - 1) * eps * math.log(eps) + _CONFIDENCE * math.log(_CONFIDENCE)

    n_tc = N - _N_SC

    R = 32
    nblocks = n_tc // R
    tc_out = pl.pallas_call(
        functools.partial(_tc_kernel, nblocks=nblocks, V=V, C=C),
        grid=(nblocks,),
        in_specs=[
            pl.BlockSpec((R, V), lambda b: (b, 0)),
            pl.BlockSpec((R, 1), lambda b: (b, 0)),
            pl.BlockSpec((R, 1), lambda b: (b, 0)),
        ],
        out_specs=pl.BlockSpec((1, 2), lambda b: (0, 0)),
        out_shape=jax.ShapeDtypeStruct((1, 2), jnp.float32),
        scratch_shapes=[pltpu.VMEM((1, 1), jnp.float32),
                        pltpu.VMEM((1, 1), jnp.float32)],
    )(inp, tgt.reshape(N, 1), m.reshape(N, 1))

    lanepad = jnp.zeros((_NW, 8), jnp.int32)
    tgt_g = jnp.repeat(tgt[n_tc:].reshape(_N_SC // 8, 8), 2, axis=0)
    m_g = jnp.repeat(m[n_tc:].reshape(_N_SC // 8, 8), 2, axis=0)
    tgt_sc = jnp.concatenate([tgt_g, lanepad], axis=1)
    m_sc = jnp.concatenate([m_g, lanepad.astype(jnp.float32)], axis=1)

    mesh = plsc.VectorSubcoreMesh(core_axis_name="c", subcore_axis_name="s")
    sc_fn = functools.partial(
        pl.kernel,
        mesh=mesh,
        out_type=jax.ShapeDtypeStruct((_NW, 2, 16), jnp.float32),
        scratch_types=[
            pltpu.VMEM((8, _CW), jnp.float32),
            pltpu.VMEM((8, _CW), jnp.float32),
            pltpu.VMEM((16,), jnp.int32),
            pltpu.VMEM((16,), jnp.float32),
            pltpu.VMEM((2, 16), jnp.float32),
            pltpu.SemaphoreType.DMA,
            pltpu.SemaphoreType.DMA,
        ],
    )(functools.partial(_sc_kernel, V=V, C=C, row0=n_tc))
    sc_out = sc_fn(inp, tgt_sc, m_sc)

    num = tc_out[0, 0] + jnp.sum(sc_out[:, 0, :])
    den = tc_out[0, 1] + jnp.sum(sc_out[:, 1, :])
    return num / den


# speedup vs baseline: 4.3373x; 1.0537x over previous
import functools
import math

import jax
import jax.numpy as jnp
from jax import lax
from jax.experimental import pallas as pl
from jax.experimental.pallas import tpu as pltpu
from jax.experimental.pallas import tpu_sc as plsc

_SMOOTHING = 0.1
_CONFIDENCE = 1.0 - _SMOOTHING

_NW = 32
_NC = 2
_RPW = 16


def _tc_kernel(inp_ref, mask_ref, out_ref, num_ref, den_ref, *,
               nblocks, V, C):
    b = pl.program_id(0)

    @pl.when(b == 0)
    def _init():
        num_ref[:, :] = jnp.zeros_like(num_ref)
        den_ref[:, :] = jnp.zeros_like(den_ref)

    eps = _SMOOTHING / (V - 1)
    x = inp_ref[:, :]
    col = jax.lax.broadcasted_iota(jnp.int32, x.shape, 1)
    x = jnp.where(col < V, x, 0.0)
    asum = jnp.sum(x, axis=1, keepdims=True)
    m = mask_ref[:, :]
    num_ref[:, :] += jnp.sum(m * (C - eps * asum), keepdims=True)
    den_ref[:, :] += jnp.sum(m, keepdims=True)

    @pl.when(b == nblocks - 1)
    def _finish():
        out_ref[:, 0:1] = num_ref[:, :]
        out_ref[:, 1:2] = den_ref[:, :]


def _sc_kernel(inp_ref, tgt_ref, mask_ref, out_ref, buf0, buf1, idx_v, m_v,
               res_v, sem0, sem1):
    wid = lax.axis_index("s") * _NC + lax.axis_index("c")
    base = wid * _RPW

    pltpu.sync_copy(tgt_ref.at[pl.ds(pl.multiple_of(base, 8), _RPW)], idx_v)
    pltpu.sync_copy(mask_ref.at[pl.ds(pl.multiple_of(base, 8), _RPW)], m_v)

    sems = [sem0, sem1]
    bufs = [buf0, buf1]
    tvec = idx_v[...]
    mvec = m_v[...]
    lane = lax.iota(jnp.int32, 16)

    def start(r):
        rbase = pl.multiple_of(base + (r // 8) * 8, 8)
        ct = pl.multiple_of(lax.bitwise_and(tvec[r], -128), 128)
        return pltpu.async_copy(
            inp_ref.at[pl.ds(rbase, 8), pl.ds(ct, 128)],
            bufs[r % 2], sems[r % 2])

    handles = [start(0), None]
    for r in range(_RPW):
        if r + 1 < _RPW:
            handles[(r + 1) % 2] = start(r + 1)
        handles[r % 2].wait()
        cbuf = bufs[r % 2]
        t_r = tvec[r]
        v16 = lax.bitwise_and(t_r, 15)
        vstart = pl.multiple_of(
            lax.bitwise_and(t_r, -16) - lax.bitwise_and(t_r, -128), 16)
        tv = cbuf[r % 8, pl.ds(vstart, 16)]
        hit = lane == jnp.broadcast_to(v16, (16,))
        res_v[r, :] = mvec[r] * jnp.where(hit, tv, 0.0)
    pltpu.sync_copy(res_v, out_ref.at[wid])


def kernel(input, target, mask):
    S = input.shape[1]
    V = input.shape[-1]
    target = target[:, :S]
    mask = mask[:, :S]
    inp = input.reshape(-1, V)
    N = inp.shape[0]
    tgt = target.reshape(N).astype(jnp.int32)
    m = mask.reshape(N).astype(jnp.float32)

    eps = _SMOOTHING / (V - 1)
    C = (V - 1) * eps * math.log(eps) + _CONFIDENCE * math.log(_CONFIDENCE)

    R = 32
    nblocks = N // R
    tc_out = pl.pallas_call(
        functools.partial(_tc_kernel, nblocks=nblocks, V=V, C=C),
        grid=(nblocks,),
        in_specs=[
            pl.BlockSpec((R, V), lambda b: (b, 0)),
            pl.BlockSpec((R, 1), lambda b: (b, 0)),
        ],
        out_specs=pl.BlockSpec((1, 2), lambda b: (0, 0)),
        out_shape=jax.ShapeDtypeStruct((1, 2), jnp.float32),
        scratch_shapes=[pltpu.VMEM((1, 1), jnp.float32),
                        pltpu.VMEM((1, 1), jnp.float32)],
    )(inp, m.reshape(N, 1))

    mesh = plsc.VectorSubcoreMesh(core_axis_name="c", subcore_axis_name="s")
    sc_fn = functools.partial(
        pl.kernel,
        mesh=mesh,
        out_type=jax.ShapeDtypeStruct((_NW, _RPW, 16), jnp.float32),
        scratch_types=[
            pltpu.VMEM((8, 128), jnp.float32),
            pltpu.VMEM((8, 128), jnp.float32),
            pltpu.VMEM((_RPW,), jnp.int32),
            pltpu.VMEM((_RPW,), jnp.float32),
            pltpu.VMEM((_RPW, 16), jnp.float32),
            pltpu.SemaphoreType.DMA,
            pltpu.SemaphoreType.DMA,
        ],
    )(_sc_kernel)
    sc_out = sc_fn(inp, tgt, m)

    num = tc_out[0, 0] - (_CONFIDENCE - eps) * jnp.sum(sc_out)
    den = tc_out[0, 1]
    return num / den


# speedup vs baseline: 5.4772x; 1.2628x over previous
import functools
import math

import jax
import jax.numpy as jnp
from jax.experimental import pallas as pl
from jax.experimental.pallas import tpu as pltpu

_SMOOTHING = 0.1
_CONFIDENCE = 1.0 - _SMOOTHING


def _loss_kernel(inp_ref, tgt_ref, mask_ref, out_ref, num_ref, den_ref, *,
                 nblocks, V, C):
    b = pl.program_id(0)

    @pl.when(b == 0)
    def _init():
        num_ref[:, :] = jnp.zeros_like(num_ref)
        den_ref[:, :] = jnp.zeros_like(den_ref)

    eps = _SMOOTHING / (V - 1)
    x = inp_ref[:, :]
    col = jax.lax.broadcasted_iota(jnp.int32, x.shape, 1)
    x = jnp.where(col < V, x, 0.0)
    hit = col == tgt_ref[:, :]
    asum = jnp.sum(x, axis=1, keepdims=True)
    atgt = jnp.sum(jnp.where(hit, x, 0.0), axis=1, keepdims=True)
    m = mask_ref[:, :]
    kl = C - eps * asum - (_CONFIDENCE - eps) * atgt
    num_ref[:, :] += jnp.sum(m * kl, keepdims=True)
    den_ref[:, :] += jnp.sum(m, keepdims=True)

    @pl.when(b == nblocks - 1)
    def _finish():
        out_ref[:, :] = num_ref[:, :] / den_ref[:, :]


def kernel(input, target, mask):
    S = input.shape[1]
    V = input.shape[-1]
    target = target[:, :S]
    mask = mask[:, :S]
    inp = input.reshape(-1, V)
    N = inp.shape[0]
    tgt = target.reshape(N, 1).astype(jnp.int32)
    m = mask.reshape(N, 1).astype(jnp.float32)

    eps = _SMOOTHING / (V - 1)
    C = (V - 1) * eps * math.log(eps) + _CONFIDENCE * math.log(_CONFIDENCE)

    R = 64
    nblocks = N // R

    out = pl.pallas_call(
        functools.partial(_loss_kernel, nblocks=nblocks, V=V, C=C),
        grid=(nblocks,),
        in_specs=[
            pl.BlockSpec((R, V), lambda b: (b, 0)),
            pl.BlockSpec((R, 1), lambda b: (b, 0)),
            pl.BlockSpec((R, 1), lambda b: (b, 0)),
        ],
        out_specs=pl.BlockSpec((1, 1), lambda b: (0, 0)),
        out_shape=jax.ShapeDtypeStruct((1, 1), jnp.float32),
        scratch_shapes=[pltpu.VMEM((1, 1), jnp.float32),
                        pltpu.VMEM((1, 1), jnp.float32)],
    )(inp, tgt, m)
    return out[0, 0]
